# Initial kernel scaffold; baseline (speedup 1.0000x reference)
import jax
import jax.numpy as jnp
from jax.experimental import pallas as pl


def kernel(feature_map, points, diff_w0, diff_b0, agg_w0, agg_b0, ln_g0, ln_b0, diff_w1, diff_b1, agg_w1, agg_b1, ln_g1, ln_b1, flat_w, flat_b, dim_w, dim_b):
    raise NotImplementedError("write your pallas kernel here")


# speedup vs baseline: 3.0026x; 3.0026x over previous
import jax
import jax.numpy as jnp
from jax import lax
from jax.experimental import pallas as pl

_INPUT_DIM = 1024
_NUM_INIT = 512
_SUB = (128, 32)
_NEIGH = (24, 24)
_B, _H, _W, _C = 4, 24, 24, 1024
_R = 8
_BN = _B * _R
_D2 = _INPUT_DIM + 2
_HP = jax.lax.Precision.HIGHEST


def _prep_body(dw0, aw0, db0, ab0, dw1, aw1, db1, ab1, m0_ref, c0_ref, m1_ref, c1_ref):
    for dw, aw, db, ab, m_ref, c_ref in ((dw0, aw0, db0, ab0, m0_ref, c0_ref),
                                         (dw1, aw1, db1, ab1, m1_ref, c1_ref)):
        a1 = aw[:, :_D2]
        mall = lax.dot_general(dw[...], a1, (((0,), (1,)), ((), ())),
                               precision=_HP, preferred_element_type=jnp.float32)
        m_ref[...] = mall
        c = lax.dot_general(db[...], a1, (((1,), (1,)), ((), ())),
                            precision=_HP, preferred_element_type=jnp.float32)
        c_ref[...] = c + ab[...]


def _sample_body(fmap_ref, pts_ref, out_ref):
    fmap = fmap_ref[0]
    p0 = pts_ref[0, 0, :]
    p1 = pts_ref[0, 1, :]
    gx = 2.0 * p1 - 1.0
    gy = 2.0 * p0 - 1.0
    x = (gx + 1.0) * 0.5 * (_W - 1)
    y = (gy + 1.0) * 0.5 * (_H - 1)
    x0 = jnp.floor(x)
    y0 = jnp.floor(y)
    x1 = x0 + 1.0
    y1 = y0 + 1.0
    wx1 = x - x0
    wx0 = 1.0 - wx1
    wy1 = y - y0
    wy0 = 1.0 - wy1
    col = lax.broadcasted_iota(jnp.int32, (_NUM_INIT, _H * _W), 1)
    acc = jnp.zeros((_NUM_INIT, _H * _W), jnp.float32)
    for ix, iy, w in ((x0, y0, wx0 * wy0), (x1, y0, wx1 * wy0),
                      (x0, y1, wx0 * wy1), (x1, y1, wx1 * wy1)):
        valid = ((ix >= 0) & (ix <= _W - 1) & (iy >= 0) & (iy <= _H - 1)).astype(jnp.float32)
        ixc = jnp.clip(ix, 0, _W - 1).astype(jnp.int32)
        iyc = jnp.clip(iy, 0, _H - 1).astype(jnp.int32)
        lin = iyc * _W + ixc
        wv = w * valid
        acc = acc + jnp.where(col == lin[:, None], wv[:, None], 0.0)
    out_ref[0] = jnp.dot(acc, fmap, precision=_HP, preferred_element_type=jnp.float32)


def _fps(p0, p1, n, npoint, fps_ref, c0_ref, c1_ref):
    col = lax.broadcasted_iota(jnp.int32, (_BN, n), 1)

    def body(i, carry):
        distance, farthest = carry
        fps_ref[pl.ds(i, 1), :] = farthest[None, :]
        oh = col == farthest[:, None]
        c0 = jnp.sum(jnp.where(oh, p0, 0.0), axis=1)
        c1 = jnp.sum(jnp.where(oh, p1, 0.0), axis=1)
        c0_ref[pl.ds(i, 1), :] = c0[None, :]
        c1_ref[pl.ds(i, 1), :] = c1[None, :]
        d = (p0 - c0[:, None]) ** 2 + (p1 - c1[:, None]) ** 2
        distance = jnp.minimum(distance, d)
        m = jnp.max(distance, axis=1, keepdims=True)
        farthest = jnp.min(jnp.where(distance == m, col, n), axis=1).astype(jnp.int32)
        return distance, farthest

    dist0 = jnp.full((_BN, n), 1e10, jnp.float32)
    far0 = jnp.zeros((_BN,), jnp.int32)
    lax.fori_loop(0, npoint, body, (dist0, far0))


def _knn(a0, a1, p0, p1, n, k, idx_ref):
    ab0 = a0.astype(jnp.bfloat16).astype(jnp.float32)
    ab1 = a1.astype(jnp.bfloat16).astype(jnp.float32)
    pb0 = p0.astype(jnp.bfloat16).astype(jnp.float32)
    pb1 = p1.astype(jnp.bfloat16).astype(jnp.float32)
    mm = ab0[:, :, None] * pb0[:, None, :] + ab1[:, :, None] * pb1[:, None, :]
    d = -2.0 * mm
    d = d + (a0 ** 2 + a1 ** 2)[:, :, None]
    d = d + (p0 ** 2 + p1 ** 2)[:, None, :]
    col = lax.broadcasted_iota(jnp.int32, d.shape, 2)

    def body(j, d):
        m = jnp.min(d, axis=2, keepdims=True)
        am = jnp.min(jnp.where(d == m, col, n), axis=2).astype(jnp.int32)
        idx_ref[pl.ds(j, 1)] = am[None]
        return jnp.where(col == am[:, :, None], jnp.inf, d)

    lax.fori_loop(0, k, body, d)


def _fps_knn_body(pts_ref, fps0_ref, np0_ref, np1_ref, idx0_ref,
                  fps1_ref, sp0_ref, sp1_ref, idx1_ref):
    p0 = pts_ref[:, 0, :]
    p1 = pts_ref[:, 1, :]
    _fps(p0, p1, _NUM_INIT, _SUB[0], fps0_ref, np0_ref, np1_ref)
    a0 = jnp.transpose(np0_ref[...])
    a1 = jnp.transpose(np1_ref[...])
    _knn(a0, a1, p0, p1, _NUM_INIT, _NEIGH[0], idx0_ref)
    _fps(a0, a1, _SUB[0], _SUB[1], fps1_ref, sp0_ref, sp1_ref)
    s0 = jnp.transpose(sp0_ref[...])
    s1 = jnp.transpose(sp1_ref[...])
    _knn(s0, s1, a0, a1, _SUB[0], _NEIGH[1], idx1_ref)


def _make_stage_body(n, s, k):
    def body(fea_ref, p0_ref, p1_ref, fps_ref, np0_ref, np1_ref, idx_ref,
             mall_ref, a2t_ref, const_ref, g_ref, b_ref, out_ref):
        fea = fea_ref[0]
        p0 = p0_ref[0, 0, :]
        p1 = p1_ref[0, 0, :]
        P = jnp.dot(fea, mall_ref[:_INPUT_DIM, :], precision=_HP,
                    preferred_element_type=jnp.float32)
        P = P + p0[:, None] * mall_ref[_INPUT_DIM:_INPUT_DIM + 1, :] \
              + p1[:, None] * mall_ref[_INPUT_DIM + 1:_INPUT_DIM + 2, :]
        fps = fps_ref[0, 0, :]
        ohcol = lax.broadcasted_iota(jnp.int32, (s, n), 1)
        oh_f = (ohcol == fps[:, None]).astype(jnp.float32)
        newfea = jnp.dot(oh_f, fea, precision=_HP, preferred_element_type=jnp.float32)
        newP = jnp.dot(oh_f, P, precision=_HP, preferred_element_type=jnp.float32)
        np0 = np0_ref[0, 0, :]
        np1 = np1_ref[0, 0, :]
        Q = jnp.dot(newfea, a2t_ref[:_INPUT_DIM, :], precision=_HP,
                    preferred_element_type=jnp.float32)
        Q = Q + np0[:, None] * a2t_ref[_INPUT_DIM:_INPUT_DIM + 1, :] \
              + np1[:, None] * a2t_ref[_INPUT_DIM + 1:_INPUT_DIM + 2, :]
        Dc = Q - newP + const_ref[...]
        g = g_ref[...]
        b = b_ref[...]
        def body(j, acc):
            ij = idx_ref[0, pl.ds(j, 1), :][0]
            oh = (ohcol == ij[:, None]).astype(jnp.float32)
            gk = jnp.dot(oh, P, precision=_HP, preferred_element_type=jnp.float32)
            h = jnp.maximum(gk + Dc, 0.0)
            mu = jnp.mean(h, axis=1, keepdims=True)
            var = jnp.mean((h - mu) ** 2, axis=1, keepdims=True)
            return acc + ((h - mu) / jnp.sqrt(var + 1e-5)) * g + b

        acc = lax.fori_loop(0, k, body, jnp.zeros((s, _INPUT_DIM), jnp.float32))
        out_ref[0] = acc * (1.0 / k)
    return body


def _flat_body(x_ref, w_ref, b_ref, o_ref):
    i = pl.program_id(0)
    part = lax.dot_general(x_ref[...], w_ref[...], (((1,), (1,)), ((), ())),
                           precision=_HP, preferred_element_type=jnp.float32)

    @pl.when(i == 0)
    def _():
        o_ref[...] = b_ref[...] + part

    @pl.when(i > 0)
    def _():
        o_ref[...] = o_ref[...] + part


def _dim_body(x_ref, w_ref, b_ref, o_ref):
    o_ref[...] = b_ref[...] + lax.dot_general(
        x_ref[...], w_ref[...], (((1,), (1,)), ((), ())),
        precision=_HP, preferred_element_type=jnp.float32)


def kernel(feature_map, points, diff_w0, diff_b0, agg_w0, agg_b0, ln_g0, ln_b0,
           diff_w1, diff_b1, agg_w1, agg_b1, ln_g1, ln_b1, flat_w, flat_b, dim_w, dim_b):
    f32 = jnp.float32
    i32 = jnp.int32

    mall0, c0, mall1, c1 = pl.pallas_call(
        _prep_body,
        out_shape=(jax.ShapeDtypeStruct((_D2, _INPUT_DIM), f32),
                   jax.ShapeDtypeStruct((1, _INPUT_DIM), f32),
                   jax.ShapeDtypeStruct((_D2, _INPUT_DIM), f32),
                   jax.ShapeDtypeStruct((1, _INPUT_DIM), f32)),
    )(diff_w0, agg_w0, diff_b0.reshape(1, _D2), agg_b0.reshape(1, _INPUT_DIM),
      diff_w1, agg_w1, diff_b1.reshape(1, _D2), agg_b1.reshape(1, _INPUT_DIM))

    a2t0 = jnp.transpose(agg_w0[:, _D2:])
    a2t1 = jnp.transpose(agg_w1[:, _D2:])

    pts_t = jnp.transpose(points, (0, 2, 1))
    fmap = feature_map.reshape(_B, _H * _W, _C)
    all_fea = pl.pallas_call(
        _sample_body,
        grid=(_BN,),
        in_specs=[pl.BlockSpec((1, _H * _W, _C), lambda i: (i // _R, 0, 0)),
                  pl.BlockSpec((1, 2, _NUM_INIT), lambda i: (i, 0, 0))],
        out_specs=pl.BlockSpec((1, _NUM_INIT, _C), lambda i: (i, 0, 0)),
        out_shape=jax.ShapeDtypeStruct((_BN, _NUM_INIT, _C), f32),
    )(fmap, pts_t)

    s0, s1 = _SUB
    k0, k1 = _NEIGH
    fps0, np0, np1, idx0, fps1, sp0, sp1, idx1 = pl.pallas_call(
        _fps_knn_body,
        out_shape=(jax.ShapeDtypeStruct((s0, _BN), i32),
                   jax.ShapeDtypeStruct((s0, _BN), f32),
                   jax.ShapeDtypeStruct((s0, _BN), f32),
                   jax.ShapeDtypeStruct((k0, _BN, s0), i32),
                   jax.ShapeDtypeStruct((s1, _BN), i32),
                   jax.ShapeDtypeStruct((s1, _BN), f32),
                   jax.ShapeDtypeStruct((s1, _BN), f32),
                   jax.ShapeDtypeStruct((k1, _BN, s1), i32)),
    )(pts_t)

    fps0_b = jnp.transpose(fps0).reshape(_BN, 1, s0)
    np0_b = jnp.transpose(np0).reshape(_BN, 1, s0)
    np1_b = jnp.transpose(np1).reshape(_BN, 1, s0)
    idx0_b = jnp.transpose(idx0, (1, 0, 2))
    fps1_b = jnp.transpose(fps1).reshape(_BN, 1, s1)
    sp0_b = jnp.transpose(sp0).reshape(_BN, 1, s1)
    sp1_b = jnp.transpose(sp1).reshape(_BN, 1, s1)
    idx1_b = jnp.transpose(idx1, (1, 0, 2))

    p0_b = pts_t[:, 0:1, :]
    p1_b = pts_t[:, 1:2, :]

    def run_stage(n, s, k, fea, p0b, p1b, fpsb, a0b, a1b, idxb, mall, a2t, const, g, b):
        full = lambda shp: pl.BlockSpec(shp, lambda i: tuple(0 for _ in shp))
        return pl.pallas_call(
            _make_stage_body(n, s, k),
            grid=(_BN,),
            in_specs=[pl.BlockSpec((1, n, _C), lambda i: (i, 0, 0)),
                      pl.BlockSpec((1, 1, n), lambda i: (i, 0, 0)),
                      pl.BlockSpec((1, 1, n), lambda i: (i, 0, 0)),
                      pl.BlockSpec((1, 1, s), lambda i: (i, 0, 0)),
                      pl.BlockSpec((1, 1, s), lambda i: (i, 0, 0)),
                      pl.BlockSpec((1, 1, s), lambda i: (i, 0, 0)),
                      pl.BlockSpec((1, k, s), lambda i: (i, 0, 0)),
                      full((_D2, _INPUT_DIM)),
                      full((_D2, _INPUT_DIM)),
                      full((1, _INPUT_DIM)),
                      full((1, _INPUT_DIM)),
                      full((1, _INPUT_DIM))],
            out_specs=pl.BlockSpec((1, s, _INPUT_DIM), lambda i: (i, 0, 0)),
            out_shape=jax.ShapeDtypeStruct((_BN, s, _INPUT_DIM), f32),
        )(fea, p0b, p1b, fpsb, a0b, a1b, idxb, mall, a2t, const,
          g.reshape(1, _INPUT_DIM), b.reshape(1, _INPUT_DIM))

    fea0 = run_stage(_NUM_INIT, s0, k0, all_fea, p0_b, p1_b, fps0_b, np0_b, np1_b,
                     idx0_b, mall0, a2t0, c0, ln_g0, ln_b0)
    fea1 = run_stage(s0, s1, k1, fea0, np0_b, np1_b, fps1_b, sp0_b, sp1_b,
                     idx1_b, mall1, a2t1, c1, ln_g1, ln_b1)

    x = fea1.reshape(_BN, s1 * _INPUT_DIM)
    nchunk = 16
    csz = (s1 * _INPUT_DIM) // nchunk
    y = pl.pallas_call(
        _flat_body,
        grid=(nchunk,),
        in_specs=[pl.BlockSpec((_BN, csz), lambda i: (0, i)),
                  pl.BlockSpec((_INPUT_DIM, csz), lambda i: (0, i)),
                  pl.BlockSpec((1, _INPUT_DIM), lambda i: (0, 0))],
        out_specs=pl.BlockSpec((_BN, _INPUT_DIM), lambda i: (0, 0)),
        out_shape=jax.ShapeDtypeStruct((_BN, _INPUT_DIM), f32),
    )(x, flat_w, flat_b.reshape(1, _INPUT_DIM))

    out = pl.pallas_call(
        _dim_body,
        out_shape=jax.ShapeDtypeStruct((_BN, dim_w.shape[0]), f32),
    )(y, dim_w, dim_b.reshape(1, dim_w.shape[0]))
    return out
